# Initial kernel scaffold; baseline (speedup 1.0000x reference)
import jax
import jax.numpy as jnp
from jax import lax
from jax.experimental import pallas as pl
from jax.experimental.pallas import tpu as pltpu
from jax.experimental.pallas import tpu_sc as plsc

N_USERS = 25000
N_NODES = 50000
D = 64
H = 32
N_EDGES = 800000
N_LAYERS = 3
NS = 16
L = 16

EC = 80
NE_TILE = N_EDGES // NS
NEC = NE_TILE // EC
NP = N_NODES // NS
NB = 125
NNC = NP // NB
DBUF = 3136


def _body(emb, rowi, coli, out, ybuf, x1, x2, x3,
          z_sh, erow, ecol, gbuf, nbuf, nbuf2, zbuf, dbuf, sem):
    cid = lax.axis_index("c")
    sid = lax.axis_index("s")
    nbase = sid * NP
    ebase = sid * NE_TILE
    hb = cid * N_NODES

    zeros16f = jnp.zeros((L,), jnp.float32)
    ones16f = jnp.ones((L,), jnp.float32)
    zeros16i = jnp.zeros((L,), jnp.int32)
    iota16 = lax.broadcasted_iota(jnp.int32, (L,), 0)

    def fill_rows(ref, val, nrows):
        def bdy(r, c):
            ref[r, pl.ds(0, L)] = val
            ref[r, pl.ds(L, L)] = val
            return c
        lax.fori_loop(0, nrows, bdy, 0)

    fill_rows(zbuf, zeros16f, NB)
    fill_rows(gbuf, ones16f, EC)

    def zero_my_z():
        def bdy(c, carry):
            pltpu.sync_copy(zbuf, z_sh.at[pl.ds(nbase + c * NB, NB), :])
            return carry
        lax.fori_loop(0, NNC, bdy, 0)

    zero_my_z()
    plsc.subcore_barrier()

    def deg_chunk(j, carry):
        e0 = ebase + j * EC
        pltpu.sync_copy(rowi.at[pl.ds(e0, EC)], erow)
        pltpu.sync_copy(gbuf, z_sh.at[erow], add=True)
        return carry
    lax.fori_loop(0, NEC, deg_chunk, 0)
    plsc.subcore_barrier()

    def dis_chunk(c, carry):
        pltpu.sync_copy(z_sh.at[pl.ds(nbase + c * NB, NB), :], nbuf)

        def grp(j, cc):
            b = jnp.minimum(j * L, NB - L)
            d = plsc.load_gather(nbuf, [b + iota16, zeros16i])
            di = plsc.bitcast(
                jnp.int32(0x5F3759DF) - (plsc.bitcast(d, jnp.int32) >> 1),
                jnp.float32)
            for _ in range(3):
                di = di * (1.5 - 0.5 * d * di * di)
            di = jnp.where(d > 0.0, di, 0.0)
            dbuf[pl.ds(c * NB + b, L)] = di
            return cc
        lax.fori_loop(0, (NB + L - 1) // L, grp, 0)
        return carry
    lax.fori_loop(0, NNC, dis_chunk, 0)

    zero_my_z()

    def y0_chunk(c, carry):
        r0 = hb + nbase + c * NB
        pltpu.sync_copy(emb.at[pl.ds(r0, NB), :], nbuf)

        def rowb(r, cc):
            s = plsc.load_gather(dbuf, [jnp.full((L,), c * NB + r, jnp.int32)])
            nbuf[r, pl.ds(0, L)] = nbuf[r, pl.ds(0, L)] * s
            nbuf[r, pl.ds(L, L)] = nbuf[r, pl.ds(L, L)] * s
            return cc
        lax.fori_loop(0, NB, rowb, 0)
        pltpu.sync_copy(nbuf, ybuf.at[pl.ds(r0, NB), :])
        return carry
    lax.fori_loop(0, NNC, y0_chunk, 0)
    plsc.subcore_barrier()

    xrefs = (x1, x2, x3)
    for k in range(N_LAYERS):
        def edge_chunk(j, carry):
            e0 = ebase + j * EC
            pltpu.sync_copy(rowi.at[pl.ds(e0, EC)], erow)
            pltpu.sync_copy(coli.at[pl.ds(e0, EC)], ecol)

            def off(t, cc):
                ecol[pl.ds(t * L, L)] = ecol[pl.ds(t * L, L)] + hb
                return cc
            lax.fori_loop(0, EC // L, off, 0)
            pltpu.async_copy(ybuf.at[ecol], gbuf, sem).wait()
            pltpu.sync_copy(gbuf, z_sh.at[erow], add=True)
            return carry
        lax.fori_loop(0, NEC, edge_chunk, 0)
        plsc.subcore_barrier()

        last = k == N_LAYERS - 1
        xr = xrefs[k]

        def end_chunk(c, carry, last=last, xr=xr):
            r0 = hb + nbase + c * NB
            pltpu.sync_copy(z_sh.at[pl.ds(nbase + c * NB, NB), :], nbuf)

            def rowb(r, cc):
                s = plsc.load_gather(
                    dbuf, [jnp.full((L,), c * NB + r, jnp.int32)])
                a0 = nbuf[r, pl.ds(0, L)] * s
                a1 = nbuf[r, pl.ds(L, L)] * s
                nbuf2[r, pl.ds(0, L)] = a0
                nbuf2[r, pl.ds(L, L)] = a1
                if not last:
                    nbuf[r, pl.ds(0, L)] = a0 * s
                    nbuf[r, pl.ds(L, L)] = a1 * s
                return cc
            lax.fori_loop(0, NB, rowb, 0)
            pltpu.sync_copy(nbuf2, xr.at[pl.ds(r0, NB), :])
            if not last:
                pltpu.sync_copy(nbuf, ybuf.at[pl.ds(r0, NB), :])
                pltpu.sync_copy(
                    zbuf, z_sh.at[pl.ds(nbase + c * NB, NB), :])
            return carry
        lax.fori_loop(0, NNC, end_chunk, 0)
        if not last:
            plsc.subcore_barrier()

    def fin_chunk(c, carry):
        r0 = hb + nbase + c * NB
        pltpu.sync_copy(emb.at[pl.ds(r0, NB), :], nbuf)
        for xr in (x1, x2, x3):
            pltpu.sync_copy(xr.at[pl.ds(r0, NB), :], nbuf2)

            def addrow(r, cc):
                nbuf[r, pl.ds(0, L)] = (
                    nbuf[r, pl.ds(0, L)] + nbuf2[r, pl.ds(0, L)])
                nbuf[r, pl.ds(L, L)] = (
                    nbuf[r, pl.ds(L, L)] + nbuf2[r, pl.ds(L, L)])
                return cc
            lax.fori_loop(0, NB, addrow, 0)

        def scrow(r, cc):
            nbuf[r, pl.ds(0, L)] = nbuf[r, pl.ds(0, L)] * 0.25
            nbuf[r, pl.ds(L, L)] = nbuf[r, pl.ds(L, L)] * 0.25
            return cc
        lax.fori_loop(0, NB, scrow, 0)
        pltpu.sync_copy(nbuf, out.at[pl.ds(r0, NB), :])
        return carry
    lax.fori_loop(0, NNC, fin_chunk, 0)


_S = jax.ShapeDtypeStruct
_f32 = jnp.float32

_lightgcn_sc = pl.kernel(
    _body,
    out_type=tuple(_S((2 * N_NODES, H), _f32) for _ in range(5)),
    mesh=plsc.VectorSubcoreMesh(core_axis_name="c", subcore_axis_name="s"),
    scratch_types=[
        pltpu.VMEM_SHARED((N_NODES, H), _f32),
        pltpu.VMEM((EC,), jnp.int32),
        pltpu.VMEM((EC,), jnp.int32),
        pltpu.VMEM((EC, H), _f32),
        pltpu.VMEM((NB, H), _f32),
        pltpu.VMEM((NB, H), _f32),
        pltpu.VMEM((NB, H), _f32),
        pltpu.VMEM((DBUF,), _f32),
        pltpu.SemaphoreType.DMA,
    ],
)


def kernel(emb_weight, edge_index):
    rowi = edge_index[0].astype(jnp.int32)
    coli = edge_index[1].astype(jnp.int32)
    embh = jnp.concatenate([emb_weight[:, :H], emb_weight[:, H:]], axis=0)
    outs = _lightgcn_sc(embh, rowi, coli)
    o = outs[0]
    out64 = jnp.concatenate([o[:N_NODES], o[N_NODES:]], axis=1)
    return out64[:N_USERS], out64[N_USERS:]


# speedup vs baseline: 5.3735x; 5.3735x over previous
import jax
import jax.numpy as jnp
from jax import lax
from jax.experimental import pallas as pl
from jax.experimental.pallas import tpu as pltpu
from jax.experimental.pallas import tpu_sc as plsc

N_USERS = 25000
N_NODES = 50000
D = 64
H = 32
N_EDGES = 800000
N_LAYERS = 3
NS = 16
L = 16

EC = 80
NE_TILE = N_EDGES // NS
NEC = NE_TILE // EC
NP0 = 3120
NB = 80
NNC0 = NP0 // NB
DBUF = 3200


def _body(emb, rowi, coli, out, ybuf, x1, x2, x3,
          z_sh, erow, ecol, gbuf, nbuf, nbuf2, zbuf, dbuf, sem):
    cid = lax.axis_index("c")
    sid = lax.axis_index("s")
    nbase = sid * NP0
    ebase = sid * NE_TILE
    hb = cid * N_NODES
    nch = NNC0 + (sid == NS - 1).astype(jnp.int32)

    zeros16f = jnp.zeros((L,), jnp.float32)
    ones16f = jnp.ones((L,), jnp.float32)
    iota16 = lax.broadcasted_iota(jnp.int32, (L,), 0)
    _dnums = lax.GatherDimensionNumbers(
        offset_dims=(), collapsed_slice_dims=(0,), start_index_map=(0,))

    def splat_lane(vec, lane):
        idx = jnp.full((L, 1), lane, jnp.int32)
        return lax.gather(vec, idx, _dnums, (1,),
                          mode=lax.GatherScatterMode.PROMISE_IN_BOUNDS)

    def fill_rows(ref, val, nrows):
        def bdy(r, c):
            ref[r, pl.ds(0, L)] = val
            ref[r, pl.ds(L, L)] = val
            return c
        lax.fori_loop(0, nrows, bdy, 0)

    fill_rows(zbuf, zeros16f, NB)
    fill_rows(gbuf, ones16f, EC)

    def zero_my_z():
        def bdy(c, carry):
            pltpu.sync_copy(zbuf, z_sh.at[pl.ds(nbase + c * NB, NB), :])
            return carry
        lax.fori_loop(0, nch, bdy, 0)

    zero_my_z()
    plsc.subcore_barrier()

    def deg_chunk(j, carry):
        e0 = ebase + j * EC
        pltpu.sync_copy(rowi.at[pl.ds(e0, EC)], erow)
        pltpu.sync_copy(gbuf, z_sh.at[erow], add=True)
        return carry
    lax.fori_loop(0, NEC, deg_chunk, 0)
    plsc.subcore_barrier()

    def dis_chunk(c, carry):
        pltpu.sync_copy(z_sh.at[pl.ds(nbase + c * NB, NB), :], nbuf)

        def grp(g, cc):
            b = g * L
            d = jnp.zeros((L,), jnp.float32)
            for r in range(L):
                d = jnp.where(iota16 == r, nbuf[b + r, pl.ds(0, L)], d)
            di = lax.bitcast_convert_type(
                jnp.int32(0x5F3759DF)
                - (lax.bitcast_convert_type(d, jnp.int32) >> 1),
                jnp.float32)
            for _ in range(3):
                di = di * (1.5 - 0.5 * d * di * di)
            di = jnp.where(d > 0.0, di, 0.0)
            dbuf[pl.ds(c * NB + b, L)] = di
            return cc
        lax.fori_loop(0, NB // L, grp, 0)
        return carry
    lax.fori_loop(0, nch, dis_chunk, 0)

    zero_my_z()

    def y0_chunk(c, carry):
        r0 = hb + nbase + c * NB
        pltpu.sync_copy(emb.at[pl.ds(r0, NB), :], nbuf)

        def grp(g, cc):
            b = g * L
            dvals = dbuf[pl.ds(c * NB + b, L)]
            for r in range(L):
                s = splat_lane(dvals, r)
                nbuf[b + r, pl.ds(0, L)] = nbuf[b + r, pl.ds(0, L)] * s
                nbuf[b + r, pl.ds(L, L)] = nbuf[b + r, pl.ds(L, L)] * s
            return cc
        lax.fori_loop(0, NB // L, grp, 0)
        pltpu.sync_copy(nbuf, ybuf.at[pl.ds(r0, NB), :])
        return carry
    lax.fori_loop(0, nch, y0_chunk, 0)
    plsc.subcore_barrier()

    xrefs = (x1, x2, x3)
    for k in range(N_LAYERS):
        def edge_chunk(j, carry):
            e0 = ebase + j * EC
            pltpu.sync_copy(rowi.at[pl.ds(e0, EC)], erow)
            pltpu.sync_copy(coli.at[pl.ds(e0, EC)], ecol)

            def off(t, cc):
                ecol[pl.ds(t * L, L)] = ecol[pl.ds(t * L, L)] + hb
                return cc
            lax.fori_loop(0, EC // L, off, 0)
            pltpu.async_copy(ybuf.at[ecol], gbuf, sem).wait()
            pltpu.sync_copy(gbuf, z_sh.at[erow], add=True)
            return carry
        lax.fori_loop(0, NEC, edge_chunk, 0)
        plsc.subcore_barrier()

        last = k == N_LAYERS - 1
        xr = xrefs[k]

        def end_chunk(c, carry, last=last, xr=xr):
            r0 = hb + nbase + c * NB
            pltpu.sync_copy(z_sh.at[pl.ds(nbase + c * NB, NB), :], nbuf)

            def grp(g, cc):
                b = g * L
                dvals = dbuf[pl.ds(c * NB + b, L)]
                for r in range(L):
                    s = splat_lane(dvals, r)
                    a0 = nbuf[b + r, pl.ds(0, L)] * s
                    a1 = nbuf[b + r, pl.ds(L, L)] * s
                    nbuf2[b + r, pl.ds(0, L)] = a0
                    nbuf2[b + r, pl.ds(L, L)] = a1
                    if not last:
                        nbuf[b + r, pl.ds(0, L)] = a0 * s
                        nbuf[b + r, pl.ds(L, L)] = a1 * s
                return cc
            lax.fori_loop(0, NB // L, grp, 0)
            pltpu.sync_copy(nbuf2, xr.at[pl.ds(r0, NB), :])
            if not last:
                pltpu.sync_copy(nbuf, ybuf.at[pl.ds(r0, NB), :])
                pltpu.sync_copy(
                    zbuf, z_sh.at[pl.ds(nbase + c * NB, NB), :])
            return carry
        lax.fori_loop(0, nch, end_chunk, 0)
        if not last:
            plsc.subcore_barrier()

    def fin_chunk(c, carry):
        r0 = hb + nbase + c * NB
        pltpu.sync_copy(emb.at[pl.ds(r0, NB), :], nbuf)
        for xr in (x1, x2, x3):
            pltpu.sync_copy(xr.at[pl.ds(r0, NB), :], nbuf2)

            def addrow(r, cc):
                nbuf[r, pl.ds(0, L)] = (
                    nbuf[r, pl.ds(0, L)] + nbuf2[r, pl.ds(0, L)])
                nbuf[r, pl.ds(L, L)] = (
                    nbuf[r, pl.ds(L, L)] + nbuf2[r, pl.ds(L, L)])
                return cc
            lax.fori_loop(0, NB, addrow, 0)

        def scrow(r, cc):
            nbuf[r, pl.ds(0, L)] = nbuf[r, pl.ds(0, L)] * 0.25
            nbuf[r, pl.ds(L, L)] = nbuf[r, pl.ds(L, L)] * 0.25
            return cc
        lax.fori_loop(0, NB, scrow, 0)
        pltpu.sync_copy(nbuf, out.at[pl.ds(r0, NB), :])
        return carry
    lax.fori_loop(0, nch, fin_chunk, 0)


_S = jax.ShapeDtypeStruct
_f32 = jnp.float32

_lightgcn_sc = pl.kernel(
    _body,
    out_type=tuple(_S((2 * N_NODES, H), _f32) for _ in range(5)),
    mesh=plsc.VectorSubcoreMesh(core_axis_name="c", subcore_axis_name="s"),
    compiler_params=pltpu.CompilerParams(use_tc_tiling_on_sc=False),
    scratch_types=[
        pltpu.VMEM_SHARED((N_NODES, H), _f32),
        pltpu.VMEM((EC,), jnp.int32),
        pltpu.VMEM((EC,), jnp.int32),
        pltpu.VMEM((EC, H), _f32),
        pltpu.VMEM((NB, H), _f32),
        pltpu.VMEM((NB, H), _f32),
        pltpu.VMEM((NB, H), _f32),
        pltpu.VMEM((DBUF,), _f32),
        pltpu.SemaphoreType.DMA,
    ],
)


def kernel(emb_weight, edge_index):
    rowi = edge_index[0].astype(jnp.int32)
    coli = edge_index[1].astype(jnp.int32)
    embh = jnp.concatenate([emb_weight[:, :H], emb_weight[:, H:]], axis=0)
    outs = _lightgcn_sc(embh, rowi, coli)
    o = outs[0]
    out64 = jnp.concatenate([o[:N_NODES], o[N_NODES:]], axis=1)
    return out64[:N_USERS], out64[N_USERS:]
